# baseline (speedup 1.0000x reference)
import functools

import jax
import jax.numpy as jnp
from jax import lax
from jax.experimental import pallas as pl
from jax.experimental.pallas import tpu as pltpu
from jax.experimental.pallas import tpu_sc as plsc

NC, NS = 2, 16
NW = NC * NS
BATCH = 4096
SEQ = 200
IW = 100
CHUNK = SEQ
KPC = CHUNK // IW
B = BATCH * SEQ
H = 64
NPOS = 201
TPW = B // NW
BPW = BATCH // NW
IRPW = TPW // IW
NBUF = 3
NGRP = CHUNK // 16

_mesh = plsc.VectorSubcoreMesh(core_axis_name="c", subcore_axis_name="s")


def _splat_lane(vec, l):
    idx = jnp.full((16, 1), l, jnp.int32)
    dnums = lax.GatherDimensionNumbers(
        offset_dims=(), collapsed_slice_dims=(0,), start_index_map=(0,))
    return lax.gather(vec, idx, dnums, (1,),
                      mode=lax.GatherScatterMode.PROMISE_IN_BOUNDS)


@functools.partial(
    pl.kernel,
    out_type=jax.ShapeDtypeStruct((BATCH, SEQ, H), jnp.float32),
    mesh=_mesh,
    compiler_params=pltpu.CompilerParams(
        use_tc_tiling_on_sc=False, needs_layout_passes=False),
    scratch_types=[
        pltpu.VMEM((IRPW, IW), jnp.int32),
        pltpu.VMEM((TPW + 16,), jnp.int32),
        pltpu.VMEM((NBUF, 1, CHUNK, H), jnp.float32),
        pltpu.VMEM((NPOS, H), jnp.float32),
        pltpu.SemaphoreType.DMA,
        pltpu.SemaphoreType.DMA,
        pltpu.SemaphoreType.DMA,
        pltpu.SemaphoreType.DMA,
        pltpu.SemaphoreType.DMA,
        pltpu.SemaphoreType.DMA,
    ],
)
def _embed_kernel(ids_hbm, pids_hbm, wtab_hbm, ptab_hbm, out_hbm,
                  idx_v, pidx_v, wbuf, ptab_v,
                  sem_w0, sem_w1, sem_w2, sem_o0, sem_o1, sem_o2):
    sem_w = (sem_w0, sem_w1, sem_w2)
    sem_o = (sem_o0, sem_o1, sem_o2)
    wid = lax.axis_index("s") * NC + lax.axis_index("c")
    tok0 = wid * TPW
    bat0 = wid * BPW
    pltpu.sync_copy(ids_hbm.at[pl.ds(wid * IRPW, IRPW)], idx_v)
    pltpu.sync_copy(pids_hbm.at[pl.ds(tok0, TPW)], pidx_v.at[pl.ds(0, TPW)])
    pltpu.sync_copy(ptab_hbm, ptab_v)

    lanes = lax.broadcasted_iota(jnp.int32, (16,), 0)

    def out_slice(j):
        return out_hbm.at[pl.ds(bat0 + j, 1)]

    def issue_gathers(j, q):
        for k in range(KPC):
            pltpu.async_copy(
                wtab_hbm.at[idx_v.at[j * KPC + k]],
                wbuf.at[q, 0, pl.ds(k * IW, IW)], sem_w[q])

    def drain_gathers(q):
        pltpu.make_async_copy(
            wtab_hbm.at[pl.ds(0, CHUNK)], wbuf.at[q, 0], sem_w[q]).wait()

    def drain_out(j, q):
        pltpu.make_async_copy(wbuf.at[q], out_slice(j), sem_o[q]).wait()

    def add_group(wb, base, pid, nl):
        for l in range(nl):
            t = base + l
            ps = _splat_lane(pid, l)
            for p in range(H // 16):
                sl = slice(p * 16, (p + 1) * 16)
                pv = plsc.load_gather(ptab_v, [ps, p * 16 + lanes])
                wb[t, sl] = wb[t, sl] + pv

    def add_chunk(j, q):
        wb = wbuf.at[q, 0]

        def grp_body(g, c2):
            pid = pidx_v[pl.ds(j * CHUNK + g * 16, 16)]
            add_group(wb, g * 16, pid, 16)
            return c2

        lax.fori_loop(0, NGRP, grp_body, 0)
        pid = pidx_v[pl.ds(j * CHUNK + NGRP * 16, 16)]
        add_group(wb, NGRP * 16, pid, 8)

    def stage(j, q, drain_prev_out, issue_next):
        if drain_prev_out:
            drain_out(j - 2, (q + 1) % NBUF)
        if issue_next:
            issue_gathers(j + 1, (q + 1) % NBUF)
        drain_gathers(q)
        add_chunk(j, q)
        pltpu.async_copy(wbuf.at[q], out_slice(j), sem_o[q])

    issue_gathers(0, 0)
    stage(0, 0, drain_prev_out=False, issue_next=True)
    stage(1, 1, drain_prev_out=False, issue_next=True)
    stage(2, 2, drain_prev_out=True, issue_next=True)
    stage(3, 0, drain_prev_out=True, issue_next=True)

    def super_body(g, carry):
        for dj in range(NBUF):
            j = 4 + NBUF * g + dj
            stage(j, (4 + dj) % NBUF, drain_prev_out=True, issue_next=True)
        return carry

    lax.fori_loop(0, (BPW - 5) // NBUF, super_body, 0)

    stage(BPW - 1, (BPW - 1) % NBUF, drain_prev_out=True, issue_next=False)
    drain_out(BPW - 2, (BPW - 2) % NBUF)
    drain_out(BPW - 1, (BPW - 1) % NBUF)


def kernel(input_ids, position_ids, word_embeddings, position_embeddings):
    ids = input_ids.reshape(-1).astype(jnp.int32).reshape(B // IW, IW)
    pids = position_ids.reshape(-1).astype(jnp.int32)
    return _embed_kernel(ids, pids, word_embeddings, position_embeddings)


# speedup vs baseline: 1.0016x; 1.0016x over previous
import functools

import jax
import jax.numpy as jnp
from jax import lax
from jax.experimental import pallas as pl
from jax.experimental.pallas import tpu as pltpu
from jax.experimental.pallas import tpu_sc as plsc

NC, NS = 2, 16
NW = NC * NS
BATCH = 4096
SEQ = 200
IWA, IWB = 104, 96
CHUNK = SEQ
B = BATCH * SEQ
H = 64
NPOS = 201
TPW = B // NW
BPW = BATCH // NW
NBUF = 3
NGRP = CHUNK // 16

_mesh = plsc.VectorSubcoreMesh(core_axis_name="c", subcore_axis_name="s")


def _splat_lane(vec, l):
    idx = jnp.full((16, 1), l, jnp.int32)
    dnums = lax.GatherDimensionNumbers(
        offset_dims=(), collapsed_slice_dims=(0,), start_index_map=(0,))
    return lax.gather(vec, idx, dnums, (1,),
                      mode=lax.GatherScatterMode.PROMISE_IN_BOUNDS)


@functools.partial(
    pl.kernel,
    out_type=jax.ShapeDtypeStruct((BATCH, SEQ, H), jnp.float32),
    mesh=_mesh,
    compiler_params=pltpu.CompilerParams(
        use_tc_tiling_on_sc=False, needs_layout_passes=False),
    scratch_types=[
        pltpu.VMEM((BPW, IWA), jnp.int32),
        pltpu.VMEM((BPW, IWB), jnp.int32),
        pltpu.VMEM((BPW, SEQ), jnp.int32),
        pltpu.VMEM((NBUF, 1, CHUNK, H), jnp.float32),
        pltpu.VMEM((NPOS, H), jnp.float32),
        pltpu.SemaphoreType.DMA,
        pltpu.SemaphoreType.DMA,
        pltpu.SemaphoreType.DMA,
        pltpu.SemaphoreType.DMA,
        pltpu.SemaphoreType.DMA,
        pltpu.SemaphoreType.DMA,
    ],
)
def _embed_kernel(ids_hbm, pids_hbm, wtab_hbm, ptab_hbm, out_hbm,
                  idx_a, idx_b, pidx_v, wbuf, ptab_v,
                  sem_w0, sem_w1, sem_w2, sem_o0, sem_o1, sem_o2):
    sem_w = (sem_w0, sem_w1, sem_w2)
    sem_o = (sem_o0, sem_o1, sem_o2)
    wid = lax.axis_index("s") * NC + lax.axis_index("c")
    bat0 = wid * BPW
    pltpu.sync_copy(ids_hbm.at[pl.ds(bat0, BPW), pl.ds(0, IWA)], idx_a)
    pltpu.sync_copy(ids_hbm.at[pl.ds(bat0, BPW), pl.ds(IWA, IWB)], idx_b)
    pltpu.sync_copy(pids_hbm.at[pl.ds(bat0, BPW)], pidx_v)
    pltpu.sync_copy(ptab_hbm, ptab_v)

    lanes = lax.broadcasted_iota(jnp.int32, (16,), 0)

    def out_slice(j):
        return out_hbm.at[pl.ds(bat0 + j, 1)]

    def issue_gathers(j, q):
        pltpu.async_copy(
            wtab_hbm.at[idx_a.at[j]],
            wbuf.at[q, 0, pl.ds(0, IWA)], sem_w[q])
        pltpu.async_copy(
            wtab_hbm.at[idx_b.at[j]],
            wbuf.at[q, 0, pl.ds(IWA, IWB)], sem_w[q])

    def drain_gathers(q):
        pltpu.make_async_copy(
            wtab_hbm.at[pl.ds(0, CHUNK)], wbuf.at[q, 0], sem_w[q]).wait()

    def drain_out(j, q):
        pltpu.make_async_copy(wbuf.at[q], out_slice(j), sem_o[q]).wait()

    def add_group(wb, base, pid, l0, l1):
        for l in range(l0, l1):
            t = base + l
            ps = _splat_lane(pid, l)
            for p in range(H // 16):
                sl = slice(p * 16, (p + 1) * 16)
                pv = plsc.load_gather(ptab_v, [ps, p * 16 + lanes])
                wb[t, sl] = wb[t, sl] + pv

    def add_chunk(j, q):
        wb = wbuf.at[q, 0]

        def grp_body(g, c2):
            pid = pidx_v[j, pl.ds(g * 16, 16)]
            add_group(wb, g * 16, pid, 0, 16)
            return c2

        lax.fori_loop(0, NGRP, grp_body, 0)
        pid = pidx_v[j, pl.ds(SEQ - 16, 16)]
        add_group(wb, SEQ - 16, pid, 8, 16)

    def stage(j, q, drain_prev_out, issue_next):
        if drain_prev_out:
            drain_out(j - 2, (q + 1) % NBUF)
        if issue_next:
            issue_gathers(j + 1, (q + 1) % NBUF)
        drain_gathers(q)
        add_chunk(j, q)
        pltpu.async_copy(wbuf.at[q], out_slice(j), sem_o[q])

    issue_gathers(0, 0)
    stage(0, 0, drain_prev_out=False, issue_next=True)
    stage(1, 1, drain_prev_out=False, issue_next=True)
    stage(2, 2, drain_prev_out=True, issue_next=True)
    stage(3, 0, drain_prev_out=True, issue_next=True)

    def super_body(g, carry):
        for dj in range(NBUF):
            j = 4 + NBUF * g + dj
            stage(j, (4 + dj) % NBUF, drain_prev_out=True, issue_next=True)
        return carry

    lax.fori_loop(0, (BPW - 5) // NBUF, super_body, 0)

    stage(BPW - 1, (BPW - 1) % NBUF, drain_prev_out=True, issue_next=False)
    drain_out(BPW - 2, (BPW - 2) % NBUF)
    drain_out(BPW - 1, (BPW - 1) % NBUF)


def kernel(input_ids, position_ids, word_embeddings, position_embeddings):
    return _embed_kernel(input_ids.astype(jnp.int32),
                         position_ids.astype(jnp.int32),
                         word_embeddings, position_embeddings)
